# Initial kernel scaffold; baseline (speedup 1.0000x reference)
import jax
import jax.numpy as jnp
from jax.experimental import pallas as pl


def kernel(values_u, values_v, values_w):
    raise NotImplementedError("write your pallas kernel here")


# speedup vs baseline: 1.1764x; 1.1764x over previous
import jax
import jax.numpy as jnp
from jax.experimental import pallas as pl
from jax.experimental.pallas import tpu as pltpu

_SIGMA = 1000000.0
_DT = 0.0005
_XMIN, _XMAX = 32, 48
_YMIN, _YMAX = 56, 72
_ZMIN, _ZMAX = 56, 72
_N = 128
_BZ = 16
_INV = 1.0 / (1.0 + _DT * _SIGMA)


def _body(u_ref, v_ref, w_ref, ou_ref, ov_ref, ow_ref):
    i = pl.program_id(0)
    z0 = i * _BZ

    zi = jax.lax.broadcasted_iota(jnp.int32, (_BZ, _N, _N), 0) + z0
    yi = jax.lax.broadcasted_iota(jnp.int32, (_BZ, _N, _N), 1)
    xi = jax.lax.broadcasted_iota(jnp.int32, (_BZ, _N, _N), 2)
    inside = (
        (zi >= _ZMIN) & (zi < _ZMAX)
        & (yi >= _YMIN) & (yi < _YMAX)
        & (xi >= _XMIN) & (xi < _XMAX)
    )
    scale = jnp.where(inside, jnp.float32(_INV), jnp.float32(1.0))
    ou_ref[...] = u_ref[...] * scale
    ov_ref[...] = v_ref[...] * scale
    ow_ref[...] = w_ref[...] * scale


def kernel(values_u, values_v, values_w):
    u3 = values_u.reshape(_N, _N, _N)
    v3 = values_v.reshape(_N, _N, _N)
    w3 = values_w.reshape(_N, _N, _N)

    spec = pl.BlockSpec((_BZ, _N, _N), lambda i: (i, 0, 0))
    out = pl.pallas_call(
        _body,
        grid=(_N // _BZ,),
        in_specs=[spec, spec, spec],
        out_specs=[spec, spec, spec],
        out_shape=[jax.ShapeDtypeStruct((_N, _N, _N), jnp.float32)] * 3,
        compiler_params=pltpu.CompilerParams(
            dimension_semantics=("arbitrary",),
        ),
    )(u3, v3, w3)
    shp = values_u.shape
    return tuple(o.reshape(shp) for o in out)


# speedup vs baseline: 1.2654x; 1.0757x over previous
import jax
import jax.numpy as jnp
from jax.experimental import pallas as pl
from jax.experimental.pallas import tpu as pltpu

_SIGMA = 1000000.0
_DT = 0.0005
_XMIN, _XMAX = 32, 48
_YMIN, _YMAX = 56, 72
_ZMIN, _ZMAX = 56, 72
_N = 128
_BZ = 32
_INV = 1.0 / (1.0 + _DT * _SIGMA)


def _body(u_ref, v_ref, w_ref, ou_ref, ov_ref, ow_ref):
    i = pl.program_id(0)
    z0 = i * _BZ
    touches = (z0 < _ZMAX) & (z0 + _BZ > _ZMIN)

    @pl.when(jnp.logical_not(touches))
    def _copy():
        ou_ref[...] = u_ref[...]
        ov_ref[...] = v_ref[...]
        ow_ref[...] = w_ref[...]

    @pl.when(touches)
    def _masked():
        zi = jax.lax.broadcasted_iota(jnp.int32, (_BZ, _N, _N), 0) + z0
        yi = jax.lax.broadcasted_iota(jnp.int32, (_BZ, _N, _N), 1)
        xi = jax.lax.broadcasted_iota(jnp.int32, (_BZ, _N, _N), 2)
        inside = (
            (zi >= _ZMIN) & (zi < _ZMAX)
            & (yi >= _YMIN) & (yi < _YMAX)
            & (xi >= _XMIN) & (xi < _XMAX)
        )
        scale = jnp.where(inside, jnp.float32(_INV), jnp.float32(1.0))
        ou_ref[...] = u_ref[...] * scale
        ov_ref[...] = v_ref[...] * scale
        ow_ref[...] = w_ref[...] * scale


def kernel(values_u, values_v, values_w):
    u3 = values_u.reshape(_N, _N, _N)
    v3 = values_v.reshape(_N, _N, _N)
    w3 = values_w.reshape(_N, _N, _N)

    spec = pl.BlockSpec((_BZ, _N, _N), lambda i: (i, 0, 0))
    out = pl.pallas_call(
        _body,
        grid=(_N // _BZ,),
        in_specs=[spec, spec, spec],
        out_specs=[spec, spec, spec],
        out_shape=[jax.ShapeDtypeStruct((_N, _N, _N), jnp.float32)] * 3,
        compiler_params=pltpu.CompilerParams(
            dimension_semantics=("arbitrary",),
        ),
    )(u3, v3, w3)
    shp = values_u.shape
    return tuple(o.reshape(shp) for o in out)


# speedup vs baseline: 1.3426x; 1.0610x over previous
import jax
import jax.numpy as jnp
from jax.experimental import pallas as pl
from jax.experimental.pallas import tpu as pltpu

_SIGMA = 1000000.0
_DT = 0.0005
_XMIN, _XMAX = 32, 48
_YMIN, _YMAX = 56, 72
_ZMIN, _ZMAX = 56, 72
_N = 128
_BZ = 64
_INV = 1.0 / (1.0 + _DT * _SIGMA)


def _body(u_ref, v_ref, w_ref, ou_ref, ov_ref, ow_ref):
    i = pl.program_id(0)
    z0 = i * _BZ
    touches = (z0 < _ZMAX) & (z0 + _BZ > _ZMIN)

    @pl.when(jnp.logical_not(touches))
    def _copy():
        ou_ref[...] = u_ref[...]
        ov_ref[...] = v_ref[...]
        ow_ref[...] = w_ref[...]

    @pl.when(touches)
    def _masked():
        zi = jax.lax.broadcasted_iota(jnp.int32, (_BZ, _N, _N), 0) + z0
        yi = jax.lax.broadcasted_iota(jnp.int32, (_BZ, _N, _N), 1)
        xi = jax.lax.broadcasted_iota(jnp.int32, (_BZ, _N, _N), 2)
        inside = (
            (zi >= _ZMIN) & (zi < _ZMAX)
            & (yi >= _YMIN) & (yi < _YMAX)
            & (xi >= _XMIN) & (xi < _XMAX)
        )
        scale = jnp.where(inside, jnp.float32(_INV), jnp.float32(1.0))
        ou_ref[...] = u_ref[...] * scale
        ov_ref[...] = v_ref[...] * scale
        ow_ref[...] = w_ref[...] * scale


def kernel(values_u, values_v, values_w):
    u3 = values_u.reshape(_N, _N, _N)
    v3 = values_v.reshape(_N, _N, _N)
    w3 = values_w.reshape(_N, _N, _N)

    spec = pl.BlockSpec((_BZ, _N, _N), lambda i: (i, 0, 0))
    out = pl.pallas_call(
        _body,
        grid=(_N // _BZ,),
        in_specs=[spec, spec, spec],
        out_specs=[spec, spec, spec],
        out_shape=[jax.ShapeDtypeStruct((_N, _N, _N), jnp.float32)] * 3,
        compiler_params=pltpu.CompilerParams(
            dimension_semantics=("arbitrary",),
        ),
    )(u3, v3, w3)
    shp = values_u.shape
    return tuple(o.reshape(shp) for o in out)
